# Initial kernel scaffold; baseline (speedup 1.0000x reference)
import jax
import jax.numpy as jnp
from jax.experimental import pallas as pl


def kernel(sites, edge_idx, edge_weight, params):
    raise NotImplementedError("write your pallas kernel here")


# speedup vs baseline: 71.7190x; 71.7190x over previous
import jax
import jax.numpy as jnp
from jax.experimental import pallas as pl
from jax.experimental.pallas import tpu as pltpu

_BS = 128
_AT = 64
_EM = 2048
_HID = 128
_NF = 128
_NG = 50
_NI = 3
_CUT = 10.0
_N = _BS * _AT
_NK = _AT * _AT
_KCH = 512
_LN2 = 0.6931471805599453


def _ssp(x):
    return jnp.maximum(x, 0.0) + jnp.log1p(jnp.exp(-jnp.abs(x))) - _LN2


def _body(x_row, ew_row, keys_col,
          w1t, b1c, w2t, b2c, cw1t, cw2t, cb2c, lwt, lbc,
          embt, embb, l1t, l1b, l2t, l2b,
          out_ref):
    f32 = jnp.float32
    ew = ew_row[...]
    step = _CUT / (_NG - 1)
    coeff = -0.5 / step ** 2
    off = jax.lax.broadcasted_iota(jnp.int32, (64, _EM), 0).astype(f32) * step
    a_t = jnp.exp(coeff * (ew - off) ** 2)
    hid = _ssp(jnp.dot(w1t[...], a_t, preferred_element_type=f32) + b1c[...])
    cosc = 0.5 * (jnp.cos(ew * (jnp.pi / _CUT)) + 1.0)
    wf = []
    for i in range(_NI):
        hi = hid[i * _NF:(i + 1) * _NF, :]
        w = jnp.dot(w2t[i], hi, preferred_element_type=f32) + b2c[i]
        wf.append(w * cosc)
    wf_all = jnp.concatenate(wf, axis=0)
    keys = keys_col[...]
    schunks = []
    for c in range(_NK // _KCH):
        kio = jax.lax.broadcasted_iota(jnp.int32, (_EM, _KCH), 1) + c * _KCH
        m = (kio == keys).astype(f32)
        schunks.append(jnp.dot(wf_all, m, preferred_element_type=f32))
    s_all = jnp.concatenate(schunks, axis=1)
    h = embt[...] * x_row[...] + embb[...]
    for i in range(_NI):
        x1 = jnp.dot(cw1t[i], h, preferred_element_type=f32)
        x1r = x1.reshape(_NF, _BS, _AT)
        sr = s_all[i * _NF:(i + 1) * _NF, :].reshape(_NF, _AT, _AT)
        aggr = jax.lax.dot_general(x1r, sr, (((2,), (1,)), ((0,), (0,))),
                                   preferred_element_type=f32)
        agg = aggr.reshape(_NF, _N)
        x2 = jnp.dot(cw2t[i], agg, preferred_element_type=f32) + cb2c[i]
        x3 = _ssp(x2)
        h = h + jnp.dot(lwt[i], x3, preferred_element_type=f32) + lbc[i]
    g = jnp.dot(l1t[...], h, preferred_element_type=f32) + l1b[...]
    o = jnp.sum(g * l2t[...], axis=0, keepdims=True) + l2b[...]
    pmat = ((jax.lax.broadcasted_iota(jnp.int32, (_N, _BS), 0) // _AT)
            == jax.lax.broadcasted_iota(jnp.int32, (_N, _BS), 1)).astype(f32)
    out_ref[...] = jnp.dot(o, pmat, preferred_element_type=f32)


def kernel(sites, edge_idx, edge_weight, params):
    p = params
    blocks = p['blocks']
    f32 = jnp.float32
    x_row = sites.astype(f32).reshape(1, _N)
    ew_row = edge_weight.astype(f32).reshape(1, _EM)
    ei = edge_idx.astype(jnp.int32)
    keys_col = (ei[:, 0] * _AT + ei[:, 1]).reshape(_EM, 1)
    w1t = jnp.concatenate(
        [jnp.pad(b['mlp_w1'].T, ((0, 0), (0, 64 - _NG))) for b in blocks], axis=0)
    b1c = jnp.concatenate([b['mlp_b1'].reshape(_NF, 1) for b in blocks], axis=0)
    w2t = jnp.stack([b['mlp_w2'].T for b in blocks])
    b2c = jnp.stack([b['mlp_b2'].reshape(_NF, 1) for b in blocks])
    cw1t = jnp.stack([b['conv_w1'].T for b in blocks])
    cw2t = jnp.stack([b['conv_w2'].T for b in blocks])
    cb2c = jnp.stack([b['conv_b2'].reshape(_HID, 1) for b in blocks])
    lwt = jnp.stack([b['lin_w'].T for b in blocks])
    lbc = jnp.stack([b['lin_b'].reshape(_HID, 1) for b in blocks])
    embt = p['emb_w'].T
    embb = p['emb_b'].reshape(_HID, 1)
    l1t = p['lin1_w'].T
    l1b = p['lin1_b'].reshape(_HID // 2, 1)
    l2t = p['lin2_w']
    l2b = p['lin2_b'].reshape(1, 1)
    out = pl.pallas_call(
        _body,
        out_shape=jax.ShapeDtypeStruct((1, _BS), f32),
        compiler_params=pltpu.CompilerParams(
            vmem_limit_bytes=100 * 1024 * 1024),
    )(x_row, ew_row, keys_col,
      w1t, b1c, w2t, b2c, cw1t, cw2t, cb2c, lwt, lbc,
      embt, embb, l1t, l1b, l2t, l2b)
    return out.reshape(_BS, 1)
